# baseline (speedup 1.0000x reference)
import functools

import jax
import jax.numpy as jnp
from jax import lax
from jax.experimental import pallas as pl
from jax.experimental.pallas import tpu as pltpu
from jax.experimental.pallas import tpu_sc as plsc

_N = 10000
_E = 320000
_NC, _NS = 2, 16
_NW = _NC * _NS
_EPT = _E // _NW
_CH = 80
_NCHUNK = _EPT // _CH
_SCH = 80
_SNCHUNK = _EPT // _SCH
_NPAD = 10240
_RPT = _NPAD // _NS
_RB = 1000
_GRID = _N // _RB


def _mesh():
    return plsc.VectorSubcoreMesh(
        core_axis_name="c", subcore_axis_name="s",
        num_cores=_NC, num_subcores=_NS)


@functools.lru_cache(maxsize=None)
def _build_sc_deg():
    @functools.partial(
        pl.kernel,
        out_type=jax.ShapeDtypeStruct((_NC, _NPAD), jnp.float32),
        mesh=_mesh(),
        compiler_params=pltpu.CompilerParams(use_tc_tiling_on_sc=False),
        scratch_types=[
            pltpu.VMEM((_EPT,), jnp.int32),
            pltpu.VMEM((_CH,), jnp.float32),
            pltpu.VMEM_SHARED((_NPAD,), jnp.float32),
        ],
    )
    def _sc_deg(edge_hbm, zeros_hbm, out_hbm, dst_v, ones_v, acc_sh):
        c = lax.axis_index("c")
        s = lax.axis_index("s")
        w = s * _NC + c
        pltpu.sync_copy(zeros_hbm.at[pl.ds(s * _RPT, _RPT)],
                        acc_sh.at[pl.ds(s * _RPT, _RPT)])
        pltpu.sync_copy(edge_hbm.at[1, pl.ds(w * _EPT, _EPT)], dst_v)
        for j in range(_CH // 16):
            ones_v[pl.ds(j * 16, 16)] = jnp.ones((16,), jnp.float32)
        plsc.subcore_barrier()

        def body(i, carry):
            pltpu.sync_copy(ones_v, acc_sh.at[dst_v.at[pl.ds(i * _CH, _CH)]],
                            add=True)
            return carry

        lax.fori_loop(0, _NCHUNK, body, 0)
        plsc.subcore_barrier()
        pltpu.sync_copy(acc_sh.at[pl.ds(s * _RPT, _RPT)],
                        out_hbm.at[c, pl.ds(s * _RPT, _RPT)])

    return _sc_deg


@functools.lru_cache(maxsize=None)
def _make_sc_scatter(D):
    @functools.partial(
        pl.kernel,
        out_type=jax.ShapeDtypeStruct((_NC, _NPAD, D), jnp.float32),
        mesh=_mesh(),
        compiler_params=pltpu.CompilerParams(use_tc_tiling_on_sc=False),
        scratch_types=[
            pltpu.VMEM((_EPT,), jnp.int32),
            pltpu.VMEM((_EPT,), jnp.int32),
            pltpu.VMEM((_SCH, D), jnp.float32),
            pltpu.VMEM((_SCH, D), jnp.float32),
            pltpu.VMEM_SHARED((_NPAD, D), jnp.float32),
            pltpu.SemaphoreType.DMA,
            pltpu.SemaphoreType.DMA,
        ],
    )
    def sc_scatter(table_hbm, edge_hbm, zeros_hbm, out_hbm,
                   src_v, dst_v, rows0, rows1, acc_sh, sem0, sem1):
        c = lax.axis_index("c")
        s = lax.axis_index("s")
        w = s * _NC + c
        pltpu.async_copy(zeros_hbm.at[pl.ds(s * _RPT, _RPT)],
                         acc_sh.at[pl.ds(s * _RPT, _RPT)], sem0)
        pltpu.sync_copy(edge_hbm.at[0, pl.ds(w * _EPT, _EPT)], src_v)
        pltpu.sync_copy(edge_hbm.at[1, pl.ds(w * _EPT, _EPT)], dst_v)
        pltpu.make_async_copy(zeros_hbm.at[pl.ds(s * _RPT, _RPT)],
                              acc_sh.at[pl.ds(s * _RPT, _RPT)], sem0).wait()
        plsc.subcore_barrier()

        def gstart(i, buf, sem):
            pltpu.async_copy(
                table_hbm.at[src_v.at[pl.ds(i * _SCH, _SCH)]], buf, sem)

        def gwait(i, buf, sem):
            pltpu.make_async_copy(
                table_hbm.at[src_v.at[pl.ds(i * _SCH, _SCH)]], buf, sem).wait()

        def scat(i, buf):
            pltpu.sync_copy(
                buf, acc_sh.at[dst_v.at[pl.ds(i * _SCH, _SCH)]], add=True)

        gstart(0, rows0, sem0)

        def body(j, carry):
            i0 = 2 * j
            gstart(i0 + 1, rows1, sem1)
            gwait(i0, rows0, sem0)
            scat(i0, rows0)
            gstart(i0 + 2, rows0, sem0)
            gwait(i0 + 1, rows1, sem1)
            scat(i0 + 1, rows1)
            return carry

        lax.fori_loop(0, _SNCHUNK // 2, body, 0)
        last = _SNCHUNK - 1
        gwait(last, rows0, sem0)
        scat(last, rows0)
        plsc.subcore_barrier()
        pltpu.sync_copy(acc_sh.at[pl.ds(s * _RPT, _RPT)],
                        out_hbm.at[c, pl.ds(s * _RPT, _RPT)])

    return sc_scatter


def _tck0_body(x_ref, w_ref, hw_ref):
    hw_ref[...] = jnp.dot(x_ref[...], w_ref[...],
                          preferred_element_type=jnp.float32)


_tck0 = pl.pallas_call(
    _tck0_body,
    grid=(_GRID,),
    in_specs=[
        pl.BlockSpec((_RB, 128), lambda i: (i, 0)),
        pl.BlockSpec((128, 128), lambda i: (0, 0)),
    ],
    out_specs=pl.BlockSpec((_RB, 128), lambda i: (i, 0)),
    out_shape=jax.ShapeDtypeStruct((_N, 128), jnp.float32),
)


def _tck1_body(degT_ref, hw_ref, hws_ref, dinv_ref):
    deg = degT_ref[:, 0:1] + degT_ref[:, 1:2] + 1.0
    dinv = lax.rsqrt(deg)
    hws_ref[...] = hw_ref[...] * dinv
    dinv_ref[...] = dinv


_tck1 = pl.pallas_call(
    _tck1_body,
    grid=(_GRID,),
    in_specs=[
        pl.BlockSpec((_RB, 2), lambda i: (i, 0)),
        pl.BlockSpec((_RB, 128), lambda i: (i, 0)),
    ],
    out_specs=[
        pl.BlockSpec((_RB, 128), lambda i: (i, 0)),
        pl.BlockSpec((_RB, 1), lambda i: (i, 0)),
    ],
    out_shape=[
        jax.ShapeDtypeStruct((_N, 128), jnp.float32),
        jax.ShapeDtypeStruct((_N, 1), jnp.float32),
    ],
)


def _make_combine_matmul(dout, relu):
    def body(p_ref, hws_ref, b_ref, dinv_ref, w_ref, h_ref, hwsn_ref):
        agg = p_ref[0] + p_ref[1] + hws_ref[...]
        h = dinv_ref[...] * agg + b_ref[...]
        if relu:
            h = jnp.maximum(h, 0.0)
        h_ref[...] = h
        hwsn_ref[...] = jnp.dot(
            h, w_ref[...], preferred_element_type=jnp.float32) * dinv_ref[...]

    return pl.pallas_call(
        body,
        grid=(_GRID,),
        in_specs=[
            pl.BlockSpec((2, _RB, 128), lambda i: (0, i, 0)),
            pl.BlockSpec((_RB, 128), lambda i: (i, 0)),
            pl.BlockSpec((1, 128), lambda i: (0, 0)),
            pl.BlockSpec((_RB, 1), lambda i: (i, 0)),
            pl.BlockSpec((128, dout), lambda i: (0, 0)),
        ],
        out_specs=[
            pl.BlockSpec((_RB, 128), lambda i: (i, 0)),
            pl.BlockSpec((_RB, dout), lambda i: (i, 0)),
        ],
        out_shape=[
            jax.ShapeDtypeStruct((_N, 128), jnp.float32),
            jax.ShapeDtypeStruct((_N, dout), jnp.float32),
        ],
    )


_tck2 = _make_combine_matmul(128, relu=True)
_tck3 = _make_combine_matmul(16, relu=False)


def _tck4_body(p_ref, hwc_ref, bc_ref, dinv_ref, o_ref):
    agg = p_ref[0] + p_ref[1] + hwc_ref[...]
    o_ref[...] = dinv_ref[...] * agg + bc_ref[...]


_tck4 = pl.pallas_call(
    _tck4_body,
    grid=(_GRID,),
    in_specs=[
        pl.BlockSpec((2, _RB, 16), lambda i: (0, i, 0)),
        pl.BlockSpec((_RB, 16), lambda i: (i, 0)),
        pl.BlockSpec((1, 16), lambda i: (0, 0)),
        pl.BlockSpec((_RB, 1), lambda i: (i, 0)),
    ],
    out_specs=pl.BlockSpec((_RB, 16), lambda i: (i, 0)),
    out_shape=jax.ShapeDtypeStruct((_N, 16), jnp.float32),
)


def kernel(x, edge_index, W1, b1, W2, b2, Wc, bc):
    z128 = jnp.zeros((_NPAD, 128), jnp.float32)
    z16 = jnp.zeros((_NPAD, 16), jnp.float32)
    zdeg = jnp.zeros((_NPAD,), jnp.float32)

    sc_deg = _build_sc_deg()
    sc_scatter128 = _make_sc_scatter(128)
    sc_scatter16 = _make_sc_scatter(16)

    hw1 = _tck0(x, W1)
    deg_p = sc_deg(edge_index, zdeg)
    degT = deg_p.T

    hws1, dinv = _tck1(degT, hw1)
    p1 = sc_scatter128(hws1, edge_index, z128)
    h1, hws2 = _tck2(p1, hws1, b1.reshape(1, 128), dinv, W2)
    p2 = sc_scatter128(hws2, edge_index, z128)
    h2, hwc = _tck3(p2, hws2, b2.reshape(1, 128), dinv, Wc)
    pc = sc_scatter16(hwc, edge_index, z16)
    out = _tck4(pc, hwc, bc.reshape(1, 16), dinv)
    return (out, h1, h2)


# speedup vs baseline: 1.0653x; 1.0653x over previous
import functools

import jax
import jax.numpy as jnp
from jax import lax
from jax.experimental import pallas as pl
from jax.experimental.pallas import tpu as pltpu
from jax.experimental.pallas import tpu_sc as plsc

_N = 10000
_E = 320000
_NC, _NS = 2, 16
_NW = _NC * _NS
_EPT = _E // _NW
_CH = 80
_NCHUNK = _EPT // _CH
_SCH = 104
_SNCHUNK = 96
_STAIL = _EPT - _SNCHUNK * _SCH
_NPAD = 10240
_RPT = _NPAD // _NS
_RB = 1000
_GRID = _N // _RB


def _mesh():
    return plsc.VectorSubcoreMesh(
        core_axis_name="c", subcore_axis_name="s",
        num_cores=_NC, num_subcores=_NS)


@functools.lru_cache(maxsize=None)
def _build_sc_deg():
    @functools.partial(
        pl.kernel,
        out_type=jax.ShapeDtypeStruct((_NC, _NPAD), jnp.float32),
        mesh=_mesh(),
        compiler_params=pltpu.CompilerParams(use_tc_tiling_on_sc=False),
        scratch_types=[
            pltpu.VMEM((_EPT,), jnp.int32),
            pltpu.VMEM((_CH,), jnp.float32),
            pltpu.VMEM_SHARED((_NPAD,), jnp.float32),
        ],
    )
    def _sc_deg(edge_hbm, zeros_hbm, out_hbm, dst_v, ones_v, acc_sh):
        c = lax.axis_index("c")
        s = lax.axis_index("s")
        w = s * _NC + c
        pltpu.sync_copy(zeros_hbm.at[pl.ds(s * _RPT, _RPT)],
                        acc_sh.at[pl.ds(s * _RPT, _RPT)])
        pltpu.sync_copy(edge_hbm.at[1, pl.ds(w * _EPT, _EPT)], dst_v)
        for j in range(_CH // 16):
            ones_v[pl.ds(j * 16, 16)] = jnp.ones((16,), jnp.float32)
        plsc.subcore_barrier()

        def body(i, carry):
            pltpu.sync_copy(ones_v, acc_sh.at[dst_v.at[pl.ds(i * _CH, _CH)]],
                            add=True)
            return carry

        lax.fori_loop(0, _NCHUNK, body, 0)
        plsc.subcore_barrier()
        pltpu.sync_copy(acc_sh.at[pl.ds(s * _RPT, _RPT)],
                        out_hbm.at[c, pl.ds(s * _RPT, _RPT)])

    return _sc_deg


@functools.lru_cache(maxsize=None)
def _make_sc_scatter(D):
    @functools.partial(
        pl.kernel,
        out_type=jax.ShapeDtypeStruct((_NC, _NPAD, D), jnp.float32),
        mesh=_mesh(),
        compiler_params=pltpu.CompilerParams(use_tc_tiling_on_sc=False),
        scratch_types=[
            pltpu.VMEM((_EPT,), jnp.int32),
            pltpu.VMEM((_EPT,), jnp.int32),
            pltpu.VMEM((_SCH, D), jnp.float32),
            pltpu.VMEM((_SCH, D), jnp.float32),
            pltpu.VMEM((_STAIL, D), jnp.float32),
            pltpu.VMEM_SHARED((_NPAD, D), jnp.float32),
            pltpu.SemaphoreType.DMA,
            pltpu.SemaphoreType.DMA,
            pltpu.SemaphoreType.DMA,
        ],
    )
    def sc_scatter(table_hbm, edge_hbm, zeros_hbm, out_hbm,
                   src_v, dst_v, rows0, rows1, rowst, acc_sh,
                   sem0, sem1, semt):
        c = lax.axis_index("c")
        s = lax.axis_index("s")
        w = s * _NC + c
        pltpu.async_copy(zeros_hbm.at[pl.ds(s * _RPT, _RPT)],
                         acc_sh.at[pl.ds(s * _RPT, _RPT)], sem0)
        pltpu.sync_copy(edge_hbm.at[0, pl.ds(w * _EPT, _EPT)], src_v)
        pltpu.sync_copy(edge_hbm.at[1, pl.ds(w * _EPT, _EPT)], dst_v)
        pltpu.make_async_copy(zeros_hbm.at[pl.ds(s * _RPT, _RPT)],
                              acc_sh.at[pl.ds(s * _RPT, _RPT)], sem0).wait()
        plsc.subcore_barrier()

        def gstart(i, buf, sem):
            pltpu.async_copy(
                table_hbm.at[src_v.at[pl.ds(i * _SCH, _SCH)]], buf, sem)

        def gwait(i, buf, sem):
            pltpu.make_async_copy(
                table_hbm.at[src_v.at[pl.ds(i * _SCH, _SCH)]], buf, sem).wait()

        def scat(i, buf):
            pltpu.sync_copy(
                buf, acc_sh.at[dst_v.at[pl.ds(i * _SCH, _SCH)]], add=True)

        tbase = _SNCHUNK * _SCH
        gstart(0, rows0, sem0)
        pltpu.async_copy(
            table_hbm.at[src_v.at[pl.ds(tbase, _STAIL)]], rowst, semt)

        def body(j, carry):
            i0 = 2 * j
            gstart(i0 + 1, rows1, sem1)
            gwait(i0, rows0, sem0)
            scat(i0, rows0)
            gstart(i0 + 2, rows0, sem0)
            gwait(i0 + 1, rows1, sem1)
            scat(i0 + 1, rows1)
            return carry

        lax.fori_loop(0, _SNCHUNK // 2 - 1, body, 0)
        i0 = _SNCHUNK - 2
        gstart(i0 + 1, rows1, sem1)
        gwait(i0, rows0, sem0)
        scat(i0, rows0)
        gwait(i0 + 1, rows1, sem1)
        scat(i0 + 1, rows1)
        pltpu.make_async_copy(
            table_hbm.at[src_v.at[pl.ds(tbase, _STAIL)]], rowst, semt).wait()
        pltpu.sync_copy(rowst, acc_sh.at[dst_v.at[pl.ds(tbase, _STAIL)]],
                        add=True)
        plsc.subcore_barrier()
        pltpu.sync_copy(acc_sh.at[pl.ds(s * _RPT, _RPT)],
                        out_hbm.at[c, pl.ds(s * _RPT, _RPT)])

    return sc_scatter


def _tck0_body(x_ref, w_ref, hw_ref):
    hw_ref[...] = jnp.dot(x_ref[...], w_ref[...],
                          preferred_element_type=jnp.float32)


_tck0 = pl.pallas_call(
    _tck0_body,
    grid=(_GRID,),
    in_specs=[
        pl.BlockSpec((_RB, 128), lambda i: (i, 0)),
        pl.BlockSpec((128, 128), lambda i: (0, 0)),
    ],
    out_specs=pl.BlockSpec((_RB, 128), lambda i: (i, 0)),
    out_shape=jax.ShapeDtypeStruct((_N, 128), jnp.float32),
)


def _tck1_body(degT_ref, hw_ref, hws_ref, dinv_ref):
    deg = degT_ref[:, 0:1] + degT_ref[:, 1:2] + 1.0
    dinv = lax.rsqrt(deg)
    hws_ref[...] = hw_ref[...] * dinv
    dinv_ref[...] = dinv


_tck1 = pl.pallas_call(
    _tck1_body,
    grid=(_GRID,),
    in_specs=[
        pl.BlockSpec((_RB, 2), lambda i: (i, 0)),
        pl.BlockSpec((_RB, 128), lambda i: (i, 0)),
    ],
    out_specs=[
        pl.BlockSpec((_RB, 128), lambda i: (i, 0)),
        pl.BlockSpec((_RB, 1), lambda i: (i, 0)),
    ],
    out_shape=[
        jax.ShapeDtypeStruct((_N, 128), jnp.float32),
        jax.ShapeDtypeStruct((_N, 1), jnp.float32),
    ],
)


def _make_combine_matmul(dout, relu):
    def body(p_ref, hws_ref, b_ref, dinv_ref, w_ref, h_ref, hwsn_ref):
        agg = p_ref[0] + p_ref[1] + hws_ref[...]
        h = dinv_ref[...] * agg + b_ref[...]
        if relu:
            h = jnp.maximum(h, 0.0)
        h_ref[...] = h
        hwsn_ref[...] = jnp.dot(
            h, w_ref[...], preferred_element_type=jnp.float32) * dinv_ref[...]

    return pl.pallas_call(
        body,
        grid=(_GRID,),
        in_specs=[
            pl.BlockSpec((2, _RB, 128), lambda i: (0, i, 0)),
            pl.BlockSpec((_RB, 128), lambda i: (i, 0)),
            pl.BlockSpec((1, 128), lambda i: (0, 0)),
            pl.BlockSpec((_RB, 1), lambda i: (i, 0)),
            pl.BlockSpec((128, dout), lambda i: (0, 0)),
        ],
        out_specs=[
            pl.BlockSpec((_RB, 128), lambda i: (i, 0)),
            pl.BlockSpec((_RB, dout), lambda i: (i, 0)),
        ],
        out_shape=[
            jax.ShapeDtypeStruct((_N, 128), jnp.float32),
            jax.ShapeDtypeStruct((_N, dout), jnp.float32),
        ],
    )


_tck2 = _make_combine_matmul(128, relu=True)
_tck3 = _make_combine_matmul(16, relu=False)


def _tck4_body(p_ref, hwc_ref, bc_ref, dinv_ref, o_ref):
    agg = p_ref[0] + p_ref[1] + hwc_ref[...]
    o_ref[...] = dinv_ref[...] * agg + bc_ref[...]


_tck4 = pl.pallas_call(
    _tck4_body,
    grid=(_GRID,),
    in_specs=[
        pl.BlockSpec((2, _RB, 16), lambda i: (0, i, 0)),
        pl.BlockSpec((_RB, 16), lambda i: (i, 0)),
        pl.BlockSpec((1, 16), lambda i: (0, 0)),
        pl.BlockSpec((_RB, 1), lambda i: (i, 0)),
    ],
    out_specs=pl.BlockSpec((_RB, 16), lambda i: (i, 0)),
    out_shape=jax.ShapeDtypeStruct((_N, 16), jnp.float32),
)


def kernel(x, edge_index, W1, b1, W2, b2, Wc, bc):
    z128 = jnp.zeros((_NPAD, 128), jnp.float32)
    z16 = jnp.zeros((_NPAD, 16), jnp.float32)
    zdeg = jnp.zeros((_NPAD,), jnp.float32)

    sc_deg = _build_sc_deg()
    sc_scatter128 = _make_sc_scatter(128)
    sc_scatter16 = _make_sc_scatter(16)

    hw1 = _tck0(x, W1)
    deg_p = sc_deg(edge_index, zdeg)
    degT = deg_p.T

    hws1, dinv = _tck1(degT, hw1)
    p1 = sc_scatter128(hws1, edge_index, z128)
    h1, hws2 = _tck2(p1, hws1, b1.reshape(1, 128), dinv, W2)
    p2 = sc_scatter128(hws2, edge_index, z128)
    h2, hwc = _tck3(p2, hws2, b2.reshape(1, 128), dinv, Wc)
    pc = sc_scatter16(hwc, edge_index, z16)
    out = _tck4(pc, hwc, bc.reshape(1, 16), dinv)
    return (out, h1, h2)


# speedup vs baseline: 1.0735x; 1.0076x over previous
import functools

import jax
import jax.numpy as jnp
from jax import lax
from jax.experimental import pallas as pl
from jax.experimental.pallas import tpu as pltpu
from jax.experimental.pallas import tpu_sc as plsc

_N = 10000
_E = 320000
_NC, _NS = 2, 16
_NW = _NC * _NS
_EPT = _E // _NW
_CH = 80
_NCHUNK = _EPT // _CH
_SCH = 104
_SNCHUNK = 96
_STAIL = _EPT - _SNCHUNK * _SCH
_NPAD = 10240
_RPT = _NPAD // _NS
_RB = 1024
_GRID = (_N + _RB - 1) // _RB


def _mesh():
    return plsc.VectorSubcoreMesh(
        core_axis_name="c", subcore_axis_name="s",
        num_cores=_NC, num_subcores=_NS)


@functools.lru_cache(maxsize=None)
def _build_sc_deg():
    @functools.partial(
        pl.kernel,
        out_type=jax.ShapeDtypeStruct((8, _NPAD), jnp.float32),
        mesh=_mesh(),
        compiler_params=pltpu.CompilerParams(use_tc_tiling_on_sc=False),
        scratch_types=[
            pltpu.VMEM((_EPT,), jnp.int32),
            pltpu.VMEM((_CH,), jnp.float32),
            pltpu.VMEM_SHARED((_NPAD,), jnp.float32),
        ],
    )
    def _sc_deg(edge_hbm, zeros_hbm, out_hbm, dst_v, ones_v, acc_sh):
        c = lax.axis_index("c")
        s = lax.axis_index("s")
        w = s * _NC + c
        pltpu.sync_copy(zeros_hbm.at[pl.ds(s * _RPT, _RPT)],
                        acc_sh.at[pl.ds(s * _RPT, _RPT)])
        pltpu.sync_copy(edge_hbm.at[1, pl.ds(w * _EPT, _EPT)], dst_v)
        for j in range(_CH // 16):
            ones_v[pl.ds(j * 16, 16)] = jnp.ones((16,), jnp.float32)
        plsc.subcore_barrier()

        def body(i, carry):
            pltpu.sync_copy(ones_v, acc_sh.at[dst_v.at[pl.ds(i * _CH, _CH)]],
                            add=True)
            return carry

        lax.fori_loop(0, _NCHUNK, body, 0)
        plsc.subcore_barrier()
        pltpu.sync_copy(acc_sh.at[pl.ds(s * _RPT, _RPT)],
                        out_hbm.at[c, pl.ds(s * _RPT, _RPT)])

    return _sc_deg


@functools.lru_cache(maxsize=None)
def _make_sc_scatter(D):
    @functools.partial(
        pl.kernel,
        out_type=jax.ShapeDtypeStruct((_NC, _NPAD, D), jnp.float32),
        mesh=_mesh(),
        compiler_params=pltpu.CompilerParams(use_tc_tiling_on_sc=False),
        scratch_types=[
            pltpu.VMEM((_EPT,), jnp.int32),
            pltpu.VMEM((_EPT,), jnp.int32),
            pltpu.VMEM((_SCH, D), jnp.float32),
            pltpu.VMEM((_SCH, D), jnp.float32),
            pltpu.VMEM((_STAIL, D), jnp.float32),
            pltpu.VMEM_SHARED((_NPAD, D), jnp.float32),
            pltpu.SemaphoreType.DMA,
            pltpu.SemaphoreType.DMA,
            pltpu.SemaphoreType.DMA,
        ],
    )
    def sc_scatter(table_hbm, edge_hbm, zeros_hbm, out_hbm,
                   src_v, dst_v, rows0, rows1, rowst, acc_sh,
                   sem0, sem1, semt):
        c = lax.axis_index("c")
        s = lax.axis_index("s")
        w = s * _NC + c
        pltpu.async_copy(zeros_hbm.at[pl.ds(s * _RPT, _RPT)],
                         acc_sh.at[pl.ds(s * _RPT, _RPT)], sem0)
        pltpu.sync_copy(edge_hbm.at[0, pl.ds(w * _EPT, _EPT)], src_v)
        pltpu.sync_copy(edge_hbm.at[1, pl.ds(w * _EPT, _EPT)], dst_v)
        pltpu.make_async_copy(zeros_hbm.at[pl.ds(s * _RPT, _RPT)],
                              acc_sh.at[pl.ds(s * _RPT, _RPT)], sem0).wait()
        plsc.subcore_barrier()

        def gstart(i, buf, sem):
            pltpu.async_copy(
                table_hbm.at[src_v.at[pl.ds(i * _SCH, _SCH)]], buf, sem)

        def gwait(i, buf, sem):
            pltpu.make_async_copy(
                table_hbm.at[src_v.at[pl.ds(i * _SCH, _SCH)]], buf, sem).wait()

        def scat(i, buf):
            pltpu.sync_copy(
                buf, acc_sh.at[dst_v.at[pl.ds(i * _SCH, _SCH)]], add=True)

        tbase = _SNCHUNK * _SCH
        gstart(0, rows0, sem0)
        pltpu.async_copy(
            table_hbm.at[src_v.at[pl.ds(tbase, _STAIL)]], rowst, semt)

        def body(j, carry):
            i0 = 2 * j
            gstart(i0 + 1, rows1, sem1)
            gwait(i0, rows0, sem0)
            scat(i0, rows0)
            gstart(i0 + 2, rows0, sem0)
            gwait(i0 + 1, rows1, sem1)
            scat(i0 + 1, rows1)
            return carry

        lax.fori_loop(0, _SNCHUNK // 2 - 1, body, 0)
        i0 = _SNCHUNK - 2
        gstart(i0 + 1, rows1, sem1)
        gwait(i0, rows0, sem0)
        scat(i0, rows0)
        gwait(i0 + 1, rows1, sem1)
        scat(i0 + 1, rows1)
        pltpu.make_async_copy(
            table_hbm.at[src_v.at[pl.ds(tbase, _STAIL)]], rowst, semt).wait()
        pltpu.sync_copy(rowst, acc_sh.at[dst_v.at[pl.ds(tbase, _STAIL)]],
                        add=True)
        plsc.subcore_barrier()
        pltpu.sync_copy(acc_sh.at[pl.ds(s * _RPT, _RPT)],
                        out_hbm.at[c, pl.ds(s * _RPT, _RPT)])

    return sc_scatter


def _dinv_of(deg8_ref):
    t = jnp.transpose(deg8_ref[...], (1, 0))
    return lax.rsqrt(t[:, 0:1] + t[:, 1:2] + 1.0)


_DEG8_SPEC = pl.BlockSpec((8, _RB), lambda i: (0, i))


def _tck0_body(x_ref, w_ref, hw_ref):
    hw_ref[...] = jnp.dot(x_ref[...], w_ref[...],
                          preferred_element_type=jnp.float32)


_tck0 = pl.pallas_call(
    _tck0_body,
    grid=(_GRID,),
    in_specs=[
        pl.BlockSpec((_RB, 128), lambda i: (i, 0)),
        pl.BlockSpec((128, 128), lambda i: (0, 0)),
    ],
    out_specs=pl.BlockSpec((_RB, 128), lambda i: (i, 0)),
    out_shape=jax.ShapeDtypeStruct((_N, 128), jnp.float32),
)


def _tck1_body(deg8_ref, hw_ref, hws_ref):
    hws_ref[...] = hw_ref[...] * _dinv_of(deg8_ref)


_tck1 = pl.pallas_call(
    _tck1_body,
    grid=(_GRID,),
    in_specs=[
        _DEG8_SPEC,
        pl.BlockSpec((_RB, 128), lambda i: (i, 0)),
    ],
    out_specs=pl.BlockSpec((_RB, 128), lambda i: (i, 0)),
    out_shape=jax.ShapeDtypeStruct((_N, 128), jnp.float32),
)


def _make_combine_matmul(dout, relu):
    def body(p_ref, hws_ref, b_ref, deg8_ref, w_ref, h_ref, hwsn_ref):
        dinv = _dinv_of(deg8_ref)
        agg = p_ref[0] + p_ref[1] + hws_ref[...]
        h = dinv * agg + b_ref[...]
        if relu:
            h = jnp.maximum(h, 0.0)
        h_ref[...] = h
        hwsn_ref[...] = jnp.dot(
            h, w_ref[...], preferred_element_type=jnp.float32) * dinv

    return pl.pallas_call(
        body,
        grid=(_GRID,),
        in_specs=[
            pl.BlockSpec((2, _RB, 128), lambda i: (0, i, 0)),
            pl.BlockSpec((_RB, 128), lambda i: (i, 0)),
            pl.BlockSpec((1, 128), lambda i: (0, 0)),
            _DEG8_SPEC,
            pl.BlockSpec((128, dout), lambda i: (0, 0)),
        ],
        out_specs=[
            pl.BlockSpec((_RB, 128), lambda i: (i, 0)),
            pl.BlockSpec((_RB, dout), lambda i: (i, 0)),
        ],
        out_shape=[
            jax.ShapeDtypeStruct((_N, 128), jnp.float32),
            jax.ShapeDtypeStruct((_N, dout), jnp.float32),
        ],
    )


_tck2 = _make_combine_matmul(128, relu=True)
_tck3 = _make_combine_matmul(16, relu=False)


def _tck4_body(p_ref, hwc_ref, bc_ref, deg8_ref, o_ref):
    agg = p_ref[0] + p_ref[1] + hwc_ref[...]
    o_ref[...] = _dinv_of(deg8_ref) * agg + bc_ref[...]


_tck4 = pl.pallas_call(
    _tck4_body,
    grid=(_GRID,),
    in_specs=[
        pl.BlockSpec((2, _RB, 16), lambda i: (0, i, 0)),
        pl.BlockSpec((_RB, 16), lambda i: (i, 0)),
        pl.BlockSpec((1, 16), lambda i: (0, 0)),
        _DEG8_SPEC,
    ],
    out_specs=pl.BlockSpec((_RB, 16), lambda i: (i, 0)),
    out_shape=jax.ShapeDtypeStruct((_N, 16), jnp.float32),
)


def kernel(x, edge_index, W1, b1, W2, b2, Wc, bc):
    z128 = jnp.zeros((_NPAD, 128), jnp.float32)
    z16 = jnp.zeros((_NPAD, 16), jnp.float32)
    zdeg = jnp.zeros((_NPAD,), jnp.float32)

    sc_deg = _build_sc_deg()
    sc_scatter128 = _make_sc_scatter(128)
    sc_scatter16 = _make_sc_scatter(16)

    hw1 = _tck0(x, W1)
    deg8 = sc_deg(edge_index, zdeg)

    hws1 = _tck1(deg8, hw1)
    p1 = sc_scatter128(hws1, edge_index, z128)
    h1, hws2 = _tck2(p1, hws1, b1.reshape(1, 128), deg8, W2)
    p2 = sc_scatter128(hws2, edge_index, z128)
    h2, hwc = _tck3(p2, hws2, b2.reshape(1, 128), deg8, Wc)
    pc = sc_scatter16(hwc, edge_index, z16)
    out = _tck4(pc, hwc, bc.reshape(1, 16), deg8)
    return (out, h1, h2)


# speedup vs baseline: 1.0777x; 1.0039x over previous
import functools

import jax
import jax.numpy as jnp
from jax import lax
from jax.experimental import pallas as pl
from jax.experimental.pallas import tpu as pltpu
from jax.experimental.pallas import tpu_sc as plsc

_N = 10000
_E = 320000
_NC, _NS = 2, 16
_NW = _NC * _NS
_EPT = _E // _NW
_CH = 128
_NCHUNK = 78
_SCH = 104
_SNCHUNK = 96
_STAIL = _EPT - _SNCHUNK * _SCH
_NPAD = 10240
_RPT = _NPAD // _NS
_RB = 1024
_GRID = (_N + _RB - 1) // _RB


def _mesh():
    return plsc.VectorSubcoreMesh(
        core_axis_name="c", subcore_axis_name="s",
        num_cores=_NC, num_subcores=_NS)


@functools.lru_cache(maxsize=None)
def _build_sc_deg():
    @functools.partial(
        pl.kernel,
        out_type=jax.ShapeDtypeStruct((8, _NPAD), jnp.float32),
        mesh=_mesh(),
        compiler_params=pltpu.CompilerParams(use_tc_tiling_on_sc=False),
        scratch_types=[
            pltpu.VMEM((_EPT,), jnp.int32),
            pltpu.VMEM((_CH,), jnp.float32),
            pltpu.VMEM_SHARED((_NPAD,), jnp.float32),
        ],
    )
    def _sc_deg(edge_hbm, zeros_hbm, out_hbm, dst_v, ones_v, acc_sh):
        c = lax.axis_index("c")
        s = lax.axis_index("s")
        w = s * _NC + c
        pltpu.sync_copy(zeros_hbm, acc_sh.at[pl.ds(s * _RPT, _RPT)])
        pltpu.sync_copy(edge_hbm.at[1, pl.ds(w * _EPT, _EPT)], dst_v)
        for j in range(_CH // 16):
            ones_v[pl.ds(j * 16, 16)] = jnp.ones((16,), jnp.float32)
        plsc.subcore_barrier()

        def body(i, carry):
            pltpu.sync_copy(ones_v, acc_sh.at[dst_v.at[pl.ds(i * _CH, _CH)]],
                            add=True)
            return carry

        lax.fori_loop(0, _NCHUNK, body, 0)
        tb = _NCHUNK * _CH
        pltpu.sync_copy(ones_v.at[pl.ds(0, _EPT - tb)],
                        acc_sh.at[dst_v.at[pl.ds(tb, _EPT - tb)]], add=True)
        plsc.subcore_barrier()
        pltpu.sync_copy(acc_sh.at[pl.ds(s * _RPT, _RPT)],
                        out_hbm.at[c, pl.ds(s * _RPT, _RPT)])

    return _sc_deg


@functools.lru_cache(maxsize=None)
def _make_sc_scatter(D):
    @functools.partial(
        pl.kernel,
        out_type=jax.ShapeDtypeStruct((_NC, _NPAD, D), jnp.float32),
        mesh=_mesh(),
        compiler_params=pltpu.CompilerParams(use_tc_tiling_on_sc=False),
        scratch_types=[
            pltpu.VMEM((_EPT,), jnp.int32),
            pltpu.VMEM((_EPT,), jnp.int32),
            pltpu.VMEM((_SCH, D), jnp.float32),
            pltpu.VMEM((_SCH, D), jnp.float32),
            pltpu.VMEM((_STAIL, D), jnp.float32),
            pltpu.VMEM_SHARED((_NPAD, D), jnp.float32),
            pltpu.SemaphoreType.DMA,
            pltpu.SemaphoreType.DMA,
            pltpu.SemaphoreType.DMA,
            pltpu.SemaphoreType.DMA,
        ],
    )
    def sc_scatter(table_hbm, edge_hbm, zeros_hbm, out_hbm,
                   src_v, dst_v, rows0, rows1, rowst, acc_sh,
                   sem0, sem1, semt, semz):
        c = lax.axis_index("c")
        s = lax.axis_index("s")
        w = s * _NC + c
        pltpu.async_copy(zeros_hbm, acc_sh.at[pl.ds(s * _RPT, _RPT)], semz)
        pltpu.sync_copy(edge_hbm.at[0, pl.ds(w * _EPT, _EPT)], src_v)
        pltpu.sync_copy(edge_hbm.at[1, pl.ds(w * _EPT, _EPT)], dst_v)

        def gstart(i, buf, sem):
            pltpu.async_copy(
                table_hbm.at[src_v.at[pl.ds(i * _SCH, _SCH)]], buf, sem)

        def gwait(i, buf, sem):
            pltpu.make_async_copy(
                table_hbm.at[src_v.at[pl.ds(i * _SCH, _SCH)]], buf, sem).wait()

        def scat(i, buf):
            pltpu.sync_copy(
                buf, acc_sh.at[dst_v.at[pl.ds(i * _SCH, _SCH)]], add=True)

        tbase = _SNCHUNK * _SCH
        gstart(0, rows0, sem0)
        gstart(1, rows1, sem1)
        pltpu.async_copy(
            table_hbm.at[src_v.at[pl.ds(tbase, _STAIL)]], rowst, semt)
        pltpu.make_async_copy(
            zeros_hbm, acc_sh.at[pl.ds(s * _RPT, _RPT)], semz).wait()
        plsc.subcore_barrier()

        def body(j, carry):
            i0 = 2 * j
            gwait(i0, rows0, sem0)
            scat(i0, rows0)
            gstart(i0 + 2, rows0, sem0)
            gwait(i0 + 1, rows1, sem1)
            scat(i0 + 1, rows1)
            gstart(i0 + 3, rows1, sem1)
            return carry

        lax.fori_loop(0, _SNCHUNK // 2 - 1, body, 0)
        i0 = _SNCHUNK - 2
        gwait(i0, rows0, sem0)
        scat(i0, rows0)
        gwait(i0 + 1, rows1, sem1)
        scat(i0 + 1, rows1)
        pltpu.make_async_copy(
            table_hbm.at[src_v.at[pl.ds(tbase, _STAIL)]], rowst, semt).wait()
        pltpu.sync_copy(rowst, acc_sh.at[dst_v.at[pl.ds(tbase, _STAIL)]],
                        add=True)
        plsc.subcore_barrier()
        pltpu.sync_copy(acc_sh.at[pl.ds(s * _RPT, _RPT)],
                        out_hbm.at[c, pl.ds(s * _RPT, _RPT)])

    return sc_scatter


def _dinv_of(deg8_ref):
    t = jnp.transpose(deg8_ref[...], (1, 0))
    return lax.rsqrt(t[:, 0:1] + t[:, 1:2] + 1.0)


_DEG8_SPEC = pl.BlockSpec((8, _RB), lambda i: (0, i))


def _tck0_body(x_ref, w_ref, hw_ref):
    hw_ref[...] = jnp.dot(x_ref[...], w_ref[...],
                          preferred_element_type=jnp.float32)


_tck0 = pl.pallas_call(
    _tck0_body,
    grid=(_GRID,),
    in_specs=[
        pl.BlockSpec((_RB, 128), lambda i: (i, 0)),
        pl.BlockSpec((128, 128), lambda i: (0, 0)),
    ],
    out_specs=pl.BlockSpec((_RB, 128), lambda i: (i, 0)),
    out_shape=jax.ShapeDtypeStruct((_N, 128), jnp.float32),
)


def _tck1_body(deg8_ref, hw_ref, hws_ref):
    hws_ref[...] = hw_ref[...] * _dinv_of(deg8_ref)


_tck1 = pl.pallas_call(
    _tck1_body,
    grid=(_GRID,),
    in_specs=[
        _DEG8_SPEC,
        pl.BlockSpec((_RB, 128), lambda i: (i, 0)),
    ],
    out_specs=pl.BlockSpec((_RB, 128), lambda i: (i, 0)),
    out_shape=jax.ShapeDtypeStruct((_N, 128), jnp.float32),
)


def _make_combine_matmul(dout, relu):
    def body(p_ref, hws_ref, b_ref, deg8_ref, w_ref, h_ref, hwsn_ref):
        dinv = _dinv_of(deg8_ref)
        agg = p_ref[0] + p_ref[1] + hws_ref[...]
        h = dinv * agg + b_ref[...]
        if relu:
            h = jnp.maximum(h, 0.0)
        h_ref[...] = h
        hwsn_ref[...] = jnp.dot(
            h, w_ref[...], preferred_element_type=jnp.float32) * dinv

    return pl.pallas_call(
        body,
        grid=(_GRID,),
        in_specs=[
            pl.BlockSpec((2, _RB, 128), lambda i: (0, i, 0)),
            pl.BlockSpec((_RB, 128), lambda i: (i, 0)),
            pl.BlockSpec((1, 128), lambda i: (0, 0)),
            _DEG8_SPEC,
            pl.BlockSpec((128, dout), lambda i: (0, 0)),
        ],
        out_specs=[
            pl.BlockSpec((_RB, 128), lambda i: (i, 0)),
            pl.BlockSpec((_RB, dout), lambda i: (i, 0)),
        ],
        out_shape=[
            jax.ShapeDtypeStruct((_N, 128), jnp.float32),
            jax.ShapeDtypeStruct((_N, dout), jnp.float32),
        ],
    )


_tck2 = _make_combine_matmul(128, relu=True)
_tck3 = _make_combine_matmul(16, relu=False)


def _tck4_body(p_ref, hwc_ref, bc_ref, deg8_ref, o_ref):
    agg = p_ref[0] + p_ref[1] + hwc_ref[...]
    o_ref[...] = _dinv_of(deg8_ref) * agg + bc_ref[...]


_tck4 = pl.pallas_call(
    _tck4_body,
    grid=(_GRID,),
    in_specs=[
        pl.BlockSpec((2, _RB, 16), lambda i: (0, i, 0)),
        pl.BlockSpec((_RB, 16), lambda i: (i, 0)),
        pl.BlockSpec((1, 16), lambda i: (0, 0)),
        _DEG8_SPEC,
    ],
    out_specs=pl.BlockSpec((_RB, 16), lambda i: (i, 0)),
    out_shape=jax.ShapeDtypeStruct((_N, 16), jnp.float32),
)


def kernel(x, edge_index, W1, b1, W2, b2, Wc, bc):
    z128 = jnp.zeros((_RPT, 128), jnp.float32)
    z16 = jnp.zeros((_RPT, 16), jnp.float32)
    zdeg = jnp.zeros((_RPT,), jnp.float32)

    sc_deg = _build_sc_deg()
    sc_scatter128 = _make_sc_scatter(128)
    sc_scatter16 = _make_sc_scatter(16)

    hw1 = _tck0(x, W1)
    deg8 = sc_deg(edge_index, zdeg)

    hws1 = _tck1(deg8, hw1)
    p1 = sc_scatter128(hws1, edge_index, z128)
    h1, hws2 = _tck2(p1, hws1, b1.reshape(1, 128), deg8, W2)
    p2 = sc_scatter128(hws2, edge_index, z128)
    h2, hwc = _tck3(p2, hws2, b2.reshape(1, 128), deg8, Wc)
    pc = sc_scatter16(hwc, edge_index, z16)
    out = _tck4(pc, hwc, bc.reshape(1, 16), deg8)
    return (out, h1, h2)
